# Initial kernel scaffold; baseline (speedup 1.0000x reference)
import jax
import jax.numpy as jnp
from jax.experimental import pallas as pl


def kernel(Z, H_case, H_disease):
    raise NotImplementedError("write your pallas kernel here")


# speedup vs baseline: 1.3888x; 1.3888x over previous
import jax
import jax.numpy as jnp
from jax.experimental import pallas as pl

_CONTRACT_ROWS = (((0,), (0,)), ((), ()))


def _readout_body(z_ref, hc_ref, hd_ref, case_ref, dis_ref):
    z = z_ref[...]
    hc = hc_ref[...]
    case_mm = jax.lax.dot_general(
        hc, z, _CONTRACT_ROWS, preferred_element_type=jnp.float32
    )
    deg = jnp.clip(jnp.sum(hc, axis=0), 1e-6, None)
    case_ref[...] = case_mm / deg[:, None]
    dis_ref[...] = jax.lax.dot_general(
        hd_ref[...], z, _CONTRACT_ROWS, preferred_element_type=jnp.float32
    )


def kernel(Z, H_case, H_disease):
    n, d = Z.shape
    e = H_case.shape[1]
    tile_e = 512
    grid = (e // tile_e,)
    case_repr, disease_repr = pl.pallas_call(
        _readout_body,
        grid=grid,
        in_specs=[
            pl.BlockSpec((n, d), lambda j: (0, 0)),
            pl.BlockSpec((n, tile_e), lambda j: (0, j)),
            pl.BlockSpec((n, tile_e), lambda j: (0, j)),
        ],
        out_specs=[
            pl.BlockSpec((tile_e, d), lambda j: (j, 0)),
            pl.BlockSpec((tile_e, d), lambda j: (j, 0)),
        ],
        out_shape=[
            jax.ShapeDtypeStruct((e, d), jnp.float32),
            jax.ShapeDtypeStruct((e, d), jnp.float32),
        ],
    )(Z, H_case, H_disease)
    return (case_repr, disease_repr)


# speedup vs baseline: 1.4709x; 1.0591x over previous
import jax
import jax.numpy as jnp
from jax.experimental import pallas as pl

_CONTRACT_ROWS = (((0,), (0,)), ((), ()))


def _readout_body(z_ref, hc_ref, hd_ref, case_ref, dis_ref):
    z = z_ref[...].astype(jnp.bfloat16)
    hc = hc_ref[...]
    case_mm = jax.lax.dot_general(
        hc.astype(jnp.bfloat16), z, _CONTRACT_ROWS,
        preferred_element_type=jnp.float32,
    )
    deg = jnp.clip(jnp.sum(hc, axis=0), 1e-6, None)
    case_ref[...] = case_mm / deg[:, None]
    dis_ref[...] = jax.lax.dot_general(
        hd_ref[...].astype(jnp.bfloat16), z, _CONTRACT_ROWS,
        preferred_element_type=jnp.float32,
    )


def kernel(Z, H_case, H_disease):
    n, d = Z.shape
    e = H_case.shape[1]
    tile_e = 512
    grid = (e // tile_e,)
    case_repr, disease_repr = pl.pallas_call(
        _readout_body,
        grid=grid,
        in_specs=[
            pl.BlockSpec((n, d), lambda j: (0, 0)),
            pl.BlockSpec((n, tile_e), lambda j: (0, j)),
            pl.BlockSpec((n, tile_e), lambda j: (0, j)),
        ],
        out_specs=[
            pl.BlockSpec((tile_e, d), lambda j: (j, 0)),
            pl.BlockSpec((tile_e, d), lambda j: (j, 0)),
        ],
        out_shape=[
            jax.ShapeDtypeStruct((e, d), jnp.float32),
            jax.ShapeDtypeStruct((e, d), jnp.float32),
        ],
    )(Z, H_case, H_disease)
    return (case_repr, disease_repr)


# speedup vs baseline: 1.5536x; 1.0562x over previous
import jax
import jax.numpy as jnp
from jax.experimental import pallas as pl

_CONTRACT_ROWS = (((0,), (0,)), ((), ()))


def _readout_body(z_ref, hc_ref, hd_ref, case_ref, dis_ref):
    z = z_ref[...].astype(jnp.bfloat16)
    hc = hc_ref[...]
    case_mm = jax.lax.dot_general(
        hc.astype(jnp.bfloat16), z, _CONTRACT_ROWS,
        preferred_element_type=jnp.float32,
    )
    deg = jnp.clip(jnp.sum(hc, axis=0), 1e-6, None)
    case_ref[...] = case_mm / deg[:, None]
    dis_ref[...] = jax.lax.dot_general(
        hd_ref[...].astype(jnp.bfloat16), z, _CONTRACT_ROWS,
        preferred_element_type=jnp.float32,
    )


def kernel(Z, H_case, H_disease):
    n, d = Z.shape
    e = H_case.shape[1]
    tile_e = 256
    grid = (e // tile_e,)
    case_repr, disease_repr = pl.pallas_call(
        _readout_body,
        grid=grid,
        in_specs=[
            pl.BlockSpec((n, d), lambda j: (0, 0)),
            pl.BlockSpec((n, tile_e), lambda j: (0, j)),
            pl.BlockSpec((n, tile_e), lambda j: (0, j)),
        ],
        out_specs=[
            pl.BlockSpec((tile_e, d), lambda j: (j, 0)),
            pl.BlockSpec((tile_e, d), lambda j: (j, 0)),
        ],
        out_shape=[
            jax.ShapeDtypeStruct((e, d), jnp.float32),
            jax.ShapeDtypeStruct((e, d), jnp.float32),
        ],
    )(Z, H_case, H_disease)
    return (case_repr, disease_repr)
